# Initial kernel scaffold; baseline (speedup 1.0000x reference)
import jax
import jax.numpy as jnp
from jax.experimental import pallas as pl


def kernel(x, neigh, batch, W1, b1, W2, b2, Wo, bo):
    raise NotImplementedError("write your pallas kernel here")


# speedup vs baseline: 15.0434x; 15.0434x over previous
import functools

import jax
import jax.numpy as jnp
from jax import lax
from jax.experimental import pallas as pl
from jax.experimental.pallas import tpu as pltpu
from jax.experimental.pallas import tpu_sc as plsc

N, D, S, H, O, B = 100000, 128, 16, 128, 128, 1024
F = B + B * S
NW = 32
BW = B // NW
RW = F // NW
CH = 16
NCH = RW // CH
HZ = CH * S // 2


def _mm_body(x_ref, w_ref, o_ref):
    o_ref[...] = jnp.dot(x_ref[...], w_ref[...],
                         preferred_element_type=jnp.float32)


def _dense_z(x, w1b):
    bm = 2000
    return pl.pallas_call(
        _mm_body,
        grid=(N // bm,),
        in_specs=[
            pl.BlockSpec((bm, D), lambda i: (i, 0)),
            pl.BlockSpec((D, H), lambda i: (0, 0)),
        ],
        out_specs=pl.BlockSpec((bm, H), lambda i: (i, 0)),
        out_shape=jax.ShapeDtypeStruct((N, H), jnp.float32),
    )(x, w1b)


def _sc_gather_max(batch32, neigh32, x, z):
    mesh = plsc.VectorSubcoreMesh(core_axis_name="c", subcore_axis_name="s")

    @functools.partial(
        pl.kernel,
        out_type=[
            jax.ShapeDtypeStruct((F, D), jnp.float32),
            jax.ShapeDtypeStruct((F, H), jnp.float32),
        ],
        mesh=mesh,
        compiler_params=pltpu.CompilerParams(use_tc_tiling_on_sc=False),
        scratch_types=[
            pltpu.VMEM((BW,), jnp.int32),
            pltpu.VMEM((BW, S), jnp.int32),
            pltpu.VMEM((RW,), jnp.int32),
            pltpu.VMEM((CH,), jnp.int32),
            pltpu.VMEM((CH, S), jnp.int32),
            pltpu.VMEM((HZ,), jnp.int32),
            pltpu.VMEM((HZ,), jnp.int32),
            pltpu.VMEM((CH, D), jnp.float32),
            pltpu.VMEM((HZ, H), jnp.float32),
            pltpu.VMEM((HZ, H), jnp.float32),
            pltpu.VMEM((CH, H), jnp.float32),
            pltpu.SemaphoreType.DMA,
            pltpu.SemaphoreType.DMA,
        ],
    )
    def k(batch_hbm, neigh_hbm, x_hbm, z_hbm, xf_hbm, mf_hbm,
          bs_v, nb_v, fs_v, fi_v, nf_v, zi0_v, zi1_v,
          xc_v, zc0_v, zc1_v, mc_v, sem1, sem2):
        w = lax.axis_index("c") * 16 + lax.axis_index("s")

        pltpu.sync_copy(batch_hbm.at[pl.ds(w * BW, BW)], bs_v)
        pltpu.async_copy(neigh_hbm.at[bs_v], nb_v, sem1).wait()

        fs_v[pl.ds(0, 16)] = bs_v[pl.ds(0, 16)]
        fs_v[pl.ds(16, 16)] = bs_v[pl.ds(16, 16)]

        def flat_nb(r, carry):
            fs_v[pl.ds(BW + r * S, S)] = nb_v[r, :]
            return carry
        lax.fori_loop(0, BW, flat_nb, 0)

        def seg_half(zc, i0):
            def seg(i, carry):
                accs0 = tuple(zc[i * S, pl.ds(h * 16, 16)] for h in range(8))

                def red(r, accs):
                    return tuple(
                        jnp.maximum(a, zc[i * S + r, pl.ds(h * 16, 16)])
                        for h, a in enumerate(accs))
                accs = lax.fori_loop(1, S, red, accs0)
                for h in range(8):
                    mc_v[i0 + i, pl.ds(h * 16, 16)] = accs[h]
                return carry
            lax.fori_loop(0, CH // 2, seg, 0)

        def chunk(c, carry):
            row0 = jnp.where(c < 2, w * BW + c * CH,
                             B + w * (BW * S) + (c - 2) * CH)
            fi_v[...] = fs_v[pl.ds(c * CH, CH)]
            cp_x = pltpu.async_copy(x_hbm.at[fi_v], xc_v, sem1)
            cp_n = pltpu.async_copy(neigh_hbm.at[fi_v], nf_v, sem2)
            cp_n.wait()

            for r in range(CH // 2):
                zi0_v[pl.ds(r * S, S)] = nf_v[r, :]
                zi1_v[pl.ds(r * S, S)] = nf_v[CH // 2 + r, :]

            cp_z0 = pltpu.async_copy(z_hbm.at[zi0_v], zc0_v, sem2)
            cp_z1 = pltpu.async_copy(z_hbm.at[zi1_v], zc1_v, sem2)
            cp_x.wait()
            pltpu.sync_copy(xc_v, xf_hbm.at[pl.ds(row0, CH)])
            cp_z0.wait()
            seg_half(zc0_v, 0)
            cp_z1.wait()
            seg_half(zc1_v, CH // 2)

            pltpu.sync_copy(mc_v, mf_hbm.at[pl.ds(row0, CH)])
            return carry
        lax.fori_loop(0, NCH, chunk, 0)

    return k(batch32, neigh32, x, z)


def _tail_body(xfb_ref, xfn_ref, mfb_ref, mfn_ref, w1a_ref, b1_ref,
               w2a_ref, w2b_ref, b2_ref, wo_ref, bo_ref, o_ref):
    h1b = jnp.maximum(
        jnp.dot(xfb_ref[...], w1a_ref[...], preferred_element_type=jnp.float32)
        + b1_ref[...] + mfb_ref[...], 0.0)
    h1n = jnp.maximum(
        jnp.dot(xfn_ref[...], w1a_ref[...], preferred_element_type=jnp.float32)
        + b1_ref[...] + mfn_ref[...], 0.0)
    u = jnp.dot(h1b, w2a_ref[...],
                preferred_element_type=jnp.float32) + b2_ref[...]
    wn = jnp.dot(h1n, w2b_ref[...], preferred_element_type=jnp.float32)
    m2 = jnp.max(wn.reshape(-1, S, H), axis=1)
    o_ref[...] = jnp.dot(u + m2, wo_ref[...],
                         preferred_element_type=jnp.float32) + bo_ref[...]


def _dense_tail(xf, mf, w1a, b1, w2a, w2b, b2, wo, bo):
    bm = 64
    full = lambda i: (0, 0)
    return pl.pallas_call(
        _tail_body,
        grid=(B // bm,),
        in_specs=[
            pl.BlockSpec((bm, D), lambda i: (i, 0)),
            pl.BlockSpec((bm * S, D), lambda i: (i + 1, 0)),
            pl.BlockSpec((bm, H), lambda i: (i, 0)),
            pl.BlockSpec((bm * S, H), lambda i: (i + 1, 0)),
            pl.BlockSpec((D, H), full),
            pl.BlockSpec((1, H), full),
            pl.BlockSpec((H, H), full),
            pl.BlockSpec((H, H), full),
            pl.BlockSpec((1, H), full),
            pl.BlockSpec((H, O), full),
            pl.BlockSpec((1, O), full),
        ],
        out_specs=pl.BlockSpec((bm, O), lambda i: (i, 0)),
        out_shape=jax.ShapeDtypeStruct((B, O), jnp.float32),
    )(xf, xf, mf, mf, w1a, b1.reshape(1, H), w2a, w2b, b2.reshape(1, H),
      wo, bo.reshape(1, O))


@jax.jit
def kernel(x, neigh, batch, W1, b1, W2, b2, Wo, bo):
    neigh32 = neigh.astype(jnp.int32)
    batch32 = batch.astype(jnp.int32)
    w1a, w1b = W1[:D], W1[D:]
    w2a, w2b = W2[:H], W2[H:]

    z = _dense_z(x, w1b)
    xf, mf = _sc_gather_max(batch32, neigh32, x, z)
    return _dense_tail(xf, mf, w1a, b1, w2a, w2b, b2, Wo, bo)
